# baseline (speedup 1.0000x reference)
import jax
import jax.numpy as jnp
from jax import lax
from jax.experimental import pallas as pl
from jax.experimental.pallas import tpu as pltpu
from jax.experimental.pallas import tpu_sc as plsc

N = 10000
E = 320000
D_IN = 128
D_OUT = 64

NC = 2
NS = 16
NW = NC * NS
CH = 64
JPW = 160
E_PAD = NW * JPW * CH
STRIPE = 640
NROW = NS * STRIPE
SROWS = 40
DW = D_IN


_SC_MESH = plsc.VectorSubcoreMesh(core_axis_name="c", subcore_axis_name="s")


RB = 2
JPH = JPW // 2


def _seg_body(y_hbm, src_hbm, dst_hbm, out_hbm, srcv, dstv, rows, acc, gsem):
    c = lax.axis_index("c")
    s = lax.axis_index("s")
    wid = s * NC + (1 - c)

    def zrow(r, _):
        for q in range(D_IN // 16):
            rows[r, pl.ds(q * 16, 16)] = jnp.zeros((16,), jnp.float32)
        return 0

    lax.fori_loop(0, SROWS, zrow, 0)

    def zchunk(k, _):
        pltpu.sync_copy(rows.at[pl.ds(0, SROWS)],
                        acc.at[pl.ds(s * STRIPE + k * SROWS, SROWS)])
        return 0

    lax.fori_loop(0, STRIPE // SROWS, zchunk, 0)
    plsc.subcore_barrier()

    def gslot(j):
        return pl.ds(lax.rem(j, RB) * CH, CH)

    def half(h, _):
        base = wid * JPW + h * JPH
        pltpu.sync_copy(src_hbm.at[pl.ds(base, JPH)], srcv)
        pltpu.sync_copy(dst_hbm.at[pl.ds(base, JPH)], dstv)
        pltpu.async_copy(y_hbm.at[srcv.at[0]], rows.at[gslot(0)], gsem)

        def body(j, _):
            pltpu.make_async_copy(y_hbm.at[srcv.at[j]], rows.at[gslot(j)],
                                  gsem).wait()
            jn = lax.min(j + 1, JPH - 1)

            @pl.when(j + 1 < JPH)
            def _():
                pltpu.async_copy(y_hbm.at[srcv.at[jn]], rows.at[gslot(jn)],
                                 gsem)

            pltpu.sync_copy(rows.at[gslot(j)], acc.at[dstv.at[j]], add=True)
            return 0

        lax.fori_loop(0, JPH, body, 0)
        return 0

    lax.fori_loop(0, JPW // JPH, half, 0)
    plsc.subcore_barrier()

    def ochunk(k, _):
        r0 = s * STRIPE + k * SROWS
        pltpu.sync_copy(acc.at[pl.ds(r0, SROWS)], rows.at[pl.ds(0, SROWS)])
        pltpu.sync_copy(rows.at[pl.ds(0, SROWS)],
                        out_hbm.at[pl.ds(c * NROW + r0, SROWS)])
        return 0

    lax.fori_loop(0, STRIPE // SROWS, ochunk, 0)


_segsum = pl.kernel(
    _seg_body,
    out_type=[jax.ShapeDtypeStruct((2 * NROW, D_IN), jnp.float32)],
    mesh=_SC_MESH,
    scratch_types=[
        pltpu.VMEM((JPH, CH), jnp.int32),
        pltpu.VMEM((JPH, CH), jnp.int32),
        pltpu.VMEM((RB * CH, D_IN), jnp.float32),
        pltpu.VMEM_SHARED((NROW, D_IN), jnp.float32),
        pltpu.SemaphoreType.DMA,
    ],
    name="segsum",
)


BR = 640


def _row_spec(w):
    return pl.BlockSpec((BR, w), lambda i: (i, 0))


def _full_spec(a, b):
    return pl.BlockSpec((a, b), lambda i: (0, 0))


def _prep_body(x_ref, wl_ref, wr_ref, y_ref, z_ref):
    x = x_ref[...]
    y = jnp.dot(x, wl_ref[...], preferred_element_type=jnp.float32)
    y_ref[...] = jnp.concatenate([y, jnp.ones_like(y)], axis=1)
    z_ref[...] = jnp.dot(x, wr_ref[...], preferred_element_type=jnp.float32)


def _mean_norm(sw, d, b, z):
    s = sw[:, :D_OUT]
    dg = jnp.sum(d[:, D_OUT:], axis=1, keepdims=True) * (1.0 / D_OUT)
    dg = jnp.maximum(dg, 1.0)
    pre = s / dg + b + z
    nrm = jnp.sqrt(jnp.sum(pre * pre, axis=1, keepdims=True))
    return pre / jnp.maximum(nrm, 1e-12)


def _comb2_body(s0_ref, s1_ref, d0_ref, d1_ref, b_ref, z_ref, wl_ref, wr_ref,
                y_ref, z2_ref):
    h = _mean_norm(s0_ref[...] + s1_ref[...], d0_ref[...] + d1_ref[...],
                   b_ref[...], z_ref[...])
    y = jnp.dot(h, wl_ref[...], preferred_element_type=jnp.float32)
    y_ref[...] = jnp.concatenate([y, jnp.zeros_like(y)], axis=1)
    z2_ref[...] = jnp.dot(h, wr_ref[...], preferred_element_type=jnp.float32)


def _comb_fc_body(s0_ref, s1_ref, d0_ref, d1_ref, b_ref, z_ref, fcw_ref,
                  fcb_ref, wl_ref, wr_ref, x1_ref, y_ref, z2_ref):
    x1 = _mean_norm(s0_ref[...] + s1_ref[...], d0_ref[...] + d1_ref[...],
                    b_ref[...], z_ref[...])
    x1_ref[...] = x1
    xb = (jnp.dot(x1, fcw_ref[...], preferred_element_type=jnp.float32)
          + fcb_ref[...])
    y = jnp.dot(xb, wl_ref[...], preferred_element_type=jnp.float32)
    y_ref[...] = jnp.concatenate([y, jnp.zeros_like(y)], axis=1)
    z2_ref[...] = jnp.dot(xb, wr_ref[...], preferred_element_type=jnp.float32)


def _comb_h2_body(s0_ref, s1_ref, d0_ref, d1_ref, b_ref, z_ref, wr_ref,
                  h2_ref, z2_ref):
    h2 = _mean_norm(s0_ref[...] + s1_ref[...], d0_ref[...] + d1_ref[...],
                    b_ref[...], z_ref[...])
    h2_ref[...] = jnp.concatenate([h2, jnp.zeros_like(h2)], axis=1)
    z2_ref[...] = jnp.dot(h2, wr_ref[...], preferred_element_type=jnp.float32)


def _final_body(s0_ref, s1_ref, d0_ref, d1_ref, b_ref, z_ref, wl_ref, out_ref):
    s = (s0_ref[...] + s1_ref[...])[:, :D_OUT]
    dg = jnp.sum((d0_ref[...] + d1_ref[...])[:, D_OUT:], axis=1,
                 keepdims=True) * (1.0 / D_OUT)
    dg = jnp.maximum(dg, 1.0)
    mean = s / dg
    pre = (jnp.dot(mean, wl_ref[...], preferred_element_type=jnp.float32)
           + b_ref[...] + z_ref[...])
    nrm = jnp.sqrt(jnp.sum(pre * pre, axis=1, keepdims=True))
    out_ref[...] = pre / jnp.maximum(nrm, 1e-12)


def _grid_call(body, in_specs, out_specs, out_shapes):
    return pl.pallas_call(
        body,
        grid=(NROW // BR,),
        in_specs=in_specs,
        out_specs=out_specs,
        out_shape=out_shapes,
    )


def kernel(features1, edge_index1,
           enc1_Wl, enc1_bl, enc1_Wr,
           enc2_Wl, enc2_bl, enc2_Wr,
           fc_W, fc_b,
           dec1_Wl, dec1_bl, dec1_Wr,
           dec2_Wl, dec2_bl, dec2_Wr):
    f32 = jnp.float32
    src = edge_index1[0].astype(jnp.int32)
    dst = edge_index1[1].astype(jnp.int32)
    pad = E_PAD - E
    src2d = jnp.concatenate([src, jnp.zeros((pad,), jnp.int32)]).reshape(
        NW * JPW, CH)
    pad_dst = N + (jnp.arange(pad, dtype=jnp.int32) % (NROW - N))
    dst2d = jnp.concatenate([dst, pad_dst]).reshape(NW * JPW, CH)
    x0 = jnp.concatenate([features1, jnp.zeros((NROW - N, D_IN), f32)])

    w1l = enc1_Wl.T.astype(f32)
    w1r = enc1_Wr.T.astype(f32)
    w2l = enc2_Wl.T.astype(f32)
    w2r = enc2_Wr.T.astype(f32)
    fcw = fc_W.T.astype(f32)
    w3l = dec1_Wl.T.astype(f32)
    w3r = dec1_Wr.T.astype(f32)
    w4l = dec2_Wl.T.astype(f32)
    w4r = dec2_Wr.T.astype(f32)
    b1 = enc1_bl.reshape(1, D_OUT)
    b2 = enc2_bl.reshape(1, D_OUT)
    fcb = fc_b.reshape(1, D_OUT)
    b3 = dec1_bl.reshape(1, D_OUT)
    b4 = dec2_bl.reshape(1, D_IN)

    y1, z1 = _grid_call(
        _prep_body,
        [_row_spec(D_IN), _full_spec(D_IN, D_OUT), _full_spec(D_IN, D_OUT)],
        [_row_spec(D_IN), _row_spec(D_OUT)],
        [jax.ShapeDtypeStruct((NROW, D_IN), f32),
         jax.ShapeDtypeStruct((NROW, D_OUT), f32)],
    )(x0, w1l, w1r)

    sp = _segsum(y1, src2d, dst2d)[0]
    d0, d1 = sp[:NROW], sp[NROW:]

    deg_specs = [_row_spec(D_IN), _row_spec(D_IN),
                 _row_spec(D_IN), _row_spec(D_IN)]

    y2, z2 = _grid_call(
        _comb2_body,
        deg_specs + [_full_spec(1, D_OUT), _row_spec(D_OUT),
                     _full_spec(D_OUT, D_OUT), _full_spec(D_OUT, D_OUT)],
        [_row_spec(D_IN), _row_spec(D_OUT)],
        [jax.ShapeDtypeStruct((NROW, D_IN), f32),
         jax.ShapeDtypeStruct((NROW, D_OUT), f32)],
    )(sp[:NROW], sp[NROW:], d0, d1, b1, z1, w2l, w2r)

    sp2 = _segsum(y2, src2d, dst2d)[0]

    x1f, y3, z3 = _grid_call(
        _comb_fc_body,
        deg_specs + [_full_spec(1, D_OUT), _row_spec(D_OUT),
                     _full_spec(D_OUT, D_OUT), _full_spec(1, D_OUT),
                     _full_spec(D_OUT, D_OUT), _full_spec(D_OUT, D_OUT)],
        [_row_spec(D_OUT), _row_spec(D_IN), _row_spec(D_OUT)],
        [jax.ShapeDtypeStruct((NROW, D_OUT), f32),
         jax.ShapeDtypeStruct((NROW, D_IN), f32),
         jax.ShapeDtypeStruct((NROW, D_OUT), f32)],
    )(sp2[:NROW], sp2[NROW:], d0, d1, b2, z2, fcw, fcb, w3l, w3r)

    sp3 = _segsum(y3, src2d, dst2d)[0]

    h2, z4 = _grid_call(
        _comb_h2_body,
        deg_specs + [_full_spec(1, D_OUT), _row_spec(D_OUT),
                     _full_spec(D_OUT, D_IN)],
        [_row_spec(D_IN), _row_spec(D_IN)],
        [jax.ShapeDtypeStruct((NROW, D_IN), f32),
         jax.ShapeDtypeStruct((NROW, D_IN), f32)],
    )(sp3[:NROW], sp3[NROW:], d0, d1, b3, z3, w4r)

    sp4 = _segsum(h2, src2d, dst2d)[0]

    x1_rec = _grid_call(
        _final_body,
        deg_specs + [_full_spec(1, D_IN), _row_spec(D_IN),
                     _full_spec(D_OUT, D_IN)],
        _row_spec(D_IN),
        jax.ShapeDtypeStruct((NROW, D_IN), f32),
    )(sp4[:NROW], sp4[NROW:], d0, d1, b4, z4, w4l)

    return (x1f[:N], x1_rec[:N])


# speedup vs baseline: 2.4871x; 2.4871x over previous
import jax
import jax.numpy as jnp
from jax import lax
from jax.experimental import pallas as pl
from jax.experimental.pallas import tpu as pltpu
from jax.experimental.pallas import tpu_sc as plsc

N = 10000
E = 320000
D_IN = 128
D_OUT = 64

NC = 2
NS = 16
NW = NC * NS
CH = 64
JPW = 160
E_PAD = NW * JPW * CH
STRIPE = 640
NROW = NS * STRIPE
SROWS = 40
DW = D_IN


_SC_MESH = plsc.VectorSubcoreMesh(core_axis_name="c", subcore_axis_name="s")


RB = 2
JPH = JPW // 2


def _seg_body(y_hbm, src_hbm, dst_hbm, out_hbm, srcv, dstv, rows, acc, gsem):
    c = lax.axis_index("c")
    s = lax.axis_index("s")
    wid = s * NC + c

    def zrow(r, _):
        for q in range(D_IN // 16):
            rows[r, pl.ds(q * 16, 16)] = jnp.zeros((16,), jnp.float32)
        return 0

    lax.fori_loop(0, SROWS, zrow, 0)

    def zchunk(k, _):
        pltpu.sync_copy(rows.at[pl.ds(0, SROWS)],
                        acc.at[pl.ds(s * STRIPE + k * SROWS, SROWS)])
        return 0

    lax.fori_loop(0, STRIPE // SROWS, zchunk, 0)
    plsc.subcore_barrier()

    def gslot(j):
        return pl.ds(lax.rem(j, RB) * CH, CH)

    def half(h, _):
        base = wid * JPW + h * JPH
        pltpu.sync_copy(src_hbm.at[pl.ds(base, JPH)], srcv)
        pltpu.sync_copy(dst_hbm.at[pl.ds(base, JPH)], dstv)
        pltpu.async_copy(y_hbm.at[srcv.at[0]], rows.at[gslot(0)], gsem)

        def body(j, _):
            pltpu.make_async_copy(y_hbm.at[srcv.at[j]], rows.at[gslot(j)],
                                  gsem).wait()
            jn = lax.min(j + 1, JPH - 1)

            @pl.when(j + 1 < JPH)
            def _():
                pltpu.async_copy(y_hbm.at[srcv.at[jn]], rows.at[gslot(jn)],
                                 gsem)

            pltpu.sync_copy(rows.at[gslot(j)], acc.at[dstv.at[j]], add=True)
            return 0

        lax.fori_loop(0, JPH, body, 0)
        return 0

    lax.fori_loop(0, JPW // JPH, half, 0)
    plsc.subcore_barrier()

    def ochunk(k, _):
        r0 = s * STRIPE + k * SROWS
        pltpu.sync_copy(acc.at[pl.ds(r0, SROWS)], rows.at[pl.ds(0, SROWS)])
        pltpu.sync_copy(rows.at[pl.ds(0, SROWS)],
                        out_hbm.at[pl.ds(c * NROW + r0, SROWS)])
        return 0

    lax.fori_loop(0, STRIPE // SROWS, ochunk, 0)


_segsum = pl.kernel(
    _seg_body,
    out_type=[jax.ShapeDtypeStruct((2 * NROW, D_IN), jnp.float32)],
    mesh=_SC_MESH,
    scratch_types=[
        pltpu.VMEM((JPH, CH), jnp.int32),
        pltpu.VMEM((JPH, CH), jnp.int32),
        pltpu.VMEM((RB * CH, D_IN), jnp.float32),
        pltpu.VMEM_SHARED((NROW, D_IN), jnp.float32),
        pltpu.SemaphoreType.DMA,
    ],
    name="segsum",
)


BR = 640


def _row_spec(w):
    return pl.BlockSpec((BR, w), lambda i: (i, 0))


def _full_spec(a, b):
    return pl.BlockSpec((a, b), lambda i: (0, 0))


def _prep_body(x_ref, wl_ref, wr_ref, y_ref, z_ref):
    x = x_ref[...]
    y = jnp.dot(x, wl_ref[...], preferred_element_type=jnp.float32)
    y_ref[...] = jnp.concatenate([y, jnp.ones_like(y)], axis=1)
    z_ref[...] = jnp.dot(x, wr_ref[...], preferred_element_type=jnp.float32)


def _mean_norm(sw, d, b, z):
    s = sw[:, :D_OUT]
    dg = jnp.sum(d[:, D_OUT:], axis=1, keepdims=True) * (1.0 / D_OUT)
    dg = jnp.maximum(dg, 1.0)
    pre = s / dg + b + z
    nrm = jnp.sqrt(jnp.sum(pre * pre, axis=1, keepdims=True))
    return pre / jnp.maximum(nrm, 1e-12)


def _comb2_body(s0_ref, s1_ref, d0_ref, d1_ref, b_ref, z_ref, wl_ref, wr_ref,
                y_ref, z2_ref):
    h = _mean_norm(s0_ref[...] + s1_ref[...], d0_ref[...] + d1_ref[...],
                   b_ref[...], z_ref[...])
    y = jnp.dot(h, wl_ref[...], preferred_element_type=jnp.float32)
    y_ref[...] = jnp.concatenate([y, jnp.zeros_like(y)], axis=1)
    z2_ref[...] = jnp.dot(h, wr_ref[...], preferred_element_type=jnp.float32)


def _comb_fc_body(s0_ref, s1_ref, d0_ref, d1_ref, b_ref, z_ref, fcw_ref,
                  fcb_ref, wl_ref, wr_ref, x1_ref, y_ref, z2_ref):
    x1 = _mean_norm(s0_ref[...] + s1_ref[...], d0_ref[...] + d1_ref[...],
                    b_ref[...], z_ref[...])
    x1_ref[...] = x1
    xb = (jnp.dot(x1, fcw_ref[...], preferred_element_type=jnp.float32)
          + fcb_ref[...])
    y = jnp.dot(xb, wl_ref[...], preferred_element_type=jnp.float32)
    y_ref[...] = jnp.concatenate([y, jnp.zeros_like(y)], axis=1)
    z2_ref[...] = jnp.dot(xb, wr_ref[...], preferred_element_type=jnp.float32)


def _comb_h2_body(s0_ref, s1_ref, d0_ref, d1_ref, b_ref, z_ref, wr_ref,
                  h2_ref, z2_ref):
    h2 = _mean_norm(s0_ref[...] + s1_ref[...], d0_ref[...] + d1_ref[...],
                    b_ref[...], z_ref[...])
    h2_ref[...] = jnp.concatenate([h2, jnp.zeros_like(h2)], axis=1)
    z2_ref[...] = jnp.dot(h2, wr_ref[...], preferred_element_type=jnp.float32)


def _final_body(s0_ref, s1_ref, d0_ref, d1_ref, b_ref, z_ref, wl_ref, out_ref):
    s = (s0_ref[...] + s1_ref[...])[:, :D_OUT]
    dg = jnp.sum((d0_ref[...] + d1_ref[...])[:, D_OUT:], axis=1,
                 keepdims=True) * (1.0 / D_OUT)
    dg = jnp.maximum(dg, 1.0)
    mean = s / dg
    pre = (jnp.dot(mean, wl_ref[...], preferred_element_type=jnp.float32)
           + b_ref[...] + z_ref[...])
    nrm = jnp.sqrt(jnp.sum(pre * pre, axis=1, keepdims=True))
    out_ref[...] = pre / jnp.maximum(nrm, 1e-12)


def _grid_call(body, in_specs, out_specs, out_shapes):
    return pl.pallas_call(
        body,
        grid=(NROW // BR,),
        in_specs=in_specs,
        out_specs=out_specs,
        out_shape=out_shapes,
    )


def kernel(features1, edge_index1,
           enc1_Wl, enc1_bl, enc1_Wr,
           enc2_Wl, enc2_bl, enc2_Wr,
           fc_W, fc_b,
           dec1_Wl, dec1_bl, dec1_Wr,
           dec2_Wl, dec2_bl, dec2_Wr):
    f32 = jnp.float32
    src = edge_index1[0].astype(jnp.int32)
    dst = edge_index1[1].astype(jnp.int32)
    pad = E_PAD - E
    pad_src = jnp.arange(pad, dtype=jnp.int32) % N
    src2d = jnp.concatenate([src, pad_src]).reshape(NW * JPW, CH)
    pad_dst = N + (jnp.arange(pad, dtype=jnp.int32) % (NROW - N))
    dst2d = jnp.concatenate([dst, pad_dst]).reshape(NW * JPW, CH)
    x0 = jnp.concatenate([features1, jnp.zeros((NROW - N, D_IN), f32)])

    w1l = enc1_Wl.T.astype(f32)
    w1r = enc1_Wr.T.astype(f32)
    w2l = enc2_Wl.T.astype(f32)
    w2r = enc2_Wr.T.astype(f32)
    fcw = fc_W.T.astype(f32)
    w3l = dec1_Wl.T.astype(f32)
    w3r = dec1_Wr.T.astype(f32)
    w4l = dec2_Wl.T.astype(f32)
    w4r = dec2_Wr.T.astype(f32)
    b1 = enc1_bl.reshape(1, D_OUT)
    b2 = enc2_bl.reshape(1, D_OUT)
    fcb = fc_b.reshape(1, D_OUT)
    b3 = dec1_bl.reshape(1, D_OUT)
    b4 = dec2_bl.reshape(1, D_IN)

    y1, z1 = _grid_call(
        _prep_body,
        [_row_spec(D_IN), _full_spec(D_IN, D_OUT), _full_spec(D_IN, D_OUT)],
        [_row_spec(D_IN), _row_spec(D_OUT)],
        [jax.ShapeDtypeStruct((NROW, D_IN), f32),
         jax.ShapeDtypeStruct((NROW, D_OUT), f32)],
    )(x0, w1l, w1r)

    sp = _segsum(y1, src2d, dst2d)[0]
    d0, d1 = sp[:NROW], sp[NROW:]

    deg_specs = [_row_spec(D_IN), _row_spec(D_IN),
                 _row_spec(D_IN), _row_spec(D_IN)]

    y2, z2 = _grid_call(
        _comb2_body,
        deg_specs + [_full_spec(1, D_OUT), _row_spec(D_OUT),
                     _full_spec(D_OUT, D_OUT), _full_spec(D_OUT, D_OUT)],
        [_row_spec(D_IN), _row_spec(D_OUT)],
        [jax.ShapeDtypeStruct((NROW, D_IN), f32),
         jax.ShapeDtypeStruct((NROW, D_OUT), f32)],
    )(sp[:NROW], sp[NROW:], d0, d1, b1, z1, w2l, w2r)

    sp2 = _segsum(y2, src2d, dst2d)[0]

    x1f, y3, z3 = _grid_call(
        _comb_fc_body,
        deg_specs + [_full_spec(1, D_OUT), _row_spec(D_OUT),
                     _full_spec(D_OUT, D_OUT), _full_spec(1, D_OUT),
                     _full_spec(D_OUT, D_OUT), _full_spec(D_OUT, D_OUT)],
        [_row_spec(D_OUT), _row_spec(D_IN), _row_spec(D_OUT)],
        [jax.ShapeDtypeStruct((NROW, D_OUT), f32),
         jax.ShapeDtypeStruct((NROW, D_IN), f32),
         jax.ShapeDtypeStruct((NROW, D_OUT), f32)],
    )(sp2[:NROW], sp2[NROW:], d0, d1, b2, z2, fcw, fcb, w3l, w3r)

    sp3 = _segsum(y3, src2d, dst2d)[0]

    h2, z4 = _grid_call(
        _comb_h2_body,
        deg_specs + [_full_spec(1, D_OUT), _row_spec(D_OUT),
                     _full_spec(D_OUT, D_IN)],
        [_row_spec(D_IN), _row_spec(D_IN)],
        [jax.ShapeDtypeStruct((NROW, D_IN), f32),
         jax.ShapeDtypeStruct((NROW, D_IN), f32)],
    )(sp3[:NROW], sp3[NROW:], d0, d1, b3, z3, w4r)

    sp4 = _segsum(h2, src2d, dst2d)[0]

    x1_rec = _grid_call(
        _final_body,
        deg_specs + [_full_spec(1, D_IN), _row_spec(D_IN),
                     _full_spec(D_OUT, D_IN)],
        _row_spec(D_IN),
        jax.ShapeDtypeStruct((NROW, D_IN), f32),
    )(sp4[:NROW], sp4[NROW:], d0, d1, b4, z4, w4l)

    return (x1f[:N], x1_rec[:N])


# speedup vs baseline: 3.9032x; 1.5694x over previous
import jax
import jax.numpy as jnp
from jax import lax
from jax.experimental import pallas as pl
from jax.experimental.pallas import tpu as pltpu
from jax.experimental.pallas import tpu_sc as plsc

N = 10000
E = 320000
D_IN = 128
D_OUT = 64

NC = 2
NS = 16
NW = NC * NS
E_PAD = 327680
EPW = E_PAD // NW
STRIPE = 640
NROW = NS * STRIPE
SROWS = 40


_SC_MESH = plsc.VectorSubcoreMesh(core_axis_name="c", subcore_axis_name="s")


def _make_segsum(w, ch, rb, nhalf, name):
    jpw = EPW // ch
    jph = jpw // nhalf

    def body(y_hbm, src_hbm, dst_hbm, out_hbm, srcv, dstv, rows, acc, gsem):
        c = lax.axis_index("c")
        s = lax.axis_index("s")
        wid = s * NC + c

        def zrow(r, _):
            for q in range(w // 16):
                rows[r, pl.ds(q * 16, 16)] = jnp.zeros((16,), jnp.float32)
            return 0

        lax.fori_loop(0, SROWS, zrow, 0)

        def zchunk(k, _):
            pltpu.sync_copy(rows.at[pl.ds(0, SROWS)],
                            acc.at[pl.ds(s * STRIPE + k * SROWS, SROWS)])
            return 0

        lax.fori_loop(0, STRIPE // SROWS, zchunk, 0)
        plsc.subcore_barrier()

        def gslot(j):
            return pl.ds(lax.rem(j, rb) * ch, ch)

        def half(h, _):
            base = wid * jpw + h * jph
            pltpu.sync_copy(src_hbm.at[pl.ds(base, jph)], srcv)
            pltpu.sync_copy(dst_hbm.at[pl.ds(base, jph)], dstv)
            for j in range(rb - 1):
                pltpu.async_copy(y_hbm.at[srcv.at[j]], rows.at[gslot(j)],
                                 gsem)

            def ibody(j, _):
                pltpu.make_async_copy(y_hbm.at[srcv.at[j]],
                                      rows.at[gslot(j)], gsem).wait()
                jn = lax.min(j + rb - 1, jph - 1)

                @pl.when(j + rb - 1 < jph)
                def _():
                    pltpu.async_copy(y_hbm.at[srcv.at[jn]],
                                     rows.at[gslot(jn)], gsem)

                pltpu.sync_copy(rows.at[gslot(j)], acc.at[dstv.at[j]],
                                add=True)
                return 0

            lax.fori_loop(0, jph, ibody, 0)
            return 0

        lax.fori_loop(0, nhalf, half, 0)
        plsc.subcore_barrier()

        def ochunk(k, _):
            r0 = s * STRIPE + k * SROWS
            pltpu.sync_copy(acc.at[pl.ds(r0, SROWS)],
                            rows.at[pl.ds(0, SROWS)])
            pltpu.sync_copy(rows.at[pl.ds(0, SROWS)],
                            out_hbm.at[pl.ds(c * NROW + r0, SROWS)])
            return 0

        lax.fori_loop(0, STRIPE // SROWS, ochunk, 0)

    return pl.kernel(
        body,
        out_type=[jax.ShapeDtypeStruct((2 * NROW, w), jnp.float32)],
        mesh=_SC_MESH,
        scratch_types=[
            pltpu.VMEM((jph, ch), jnp.int32),
            pltpu.VMEM((jph, ch), jnp.int32),
            pltpu.VMEM((rb * ch, w), jnp.float32),
            pltpu.VMEM_SHARED((NROW, w), jnp.float32),
            pltpu.SemaphoreType.DMA,
        ],
        compiler_params=pltpu.CompilerParams(use_tc_tiling_on_sc=False),
        name=name,
    )


CH1 = 64
CH2 = 128
_segsum128 = _make_segsum(D_IN, CH1, 2, 2, "segsum128")
_segsum64 = _make_segsum(D_OUT, CH2, 3, 2, "segsum64")


BR = 640


def _row_spec(w):
    return pl.BlockSpec((BR, w), lambda i: (i, 0))


def _full_spec(a, b):
    return pl.BlockSpec((a, b), lambda i: (0, 0))


def _prep_body(x_ref, wl_ref, wr_ref, y_ref, z_ref):
    x = x_ref[...]
    y = jnp.dot(x, wl_ref[...], preferred_element_type=jnp.float32)
    y_ref[...] = jnp.concatenate([y, jnp.ones_like(y)], axis=1)
    z_ref[...] = jnp.dot(x, wr_ref[...], preferred_element_type=jnp.float32)


def _mean_norm(sw, d, b, z):
    s = sw[:, :D_OUT]
    dg = jnp.sum(d[:, D_OUT:], axis=1, keepdims=True) * (1.0 / D_OUT)
    dg = jnp.maximum(dg, 1.0)
    pre = s / dg + b + z
    nrm = jnp.sqrt(jnp.sum(pre * pre, axis=1, keepdims=True))
    return pre / jnp.maximum(nrm, 1e-12)


def _comb2_body(s0_ref, s1_ref, b_ref, z_ref, wl_ref, wr_ref,
                y_ref, z2_ref):
    sw = s0_ref[...] + s1_ref[...]
    h = _mean_norm(sw, sw, b_ref[...], z_ref[...])
    y_ref[...] = jnp.dot(h, wl_ref[...], preferred_element_type=jnp.float32)
    z2_ref[...] = jnp.dot(h, wr_ref[...], preferred_element_type=jnp.float32)


def _comb_fc_body(s0_ref, s1_ref, d0_ref, d1_ref, b_ref, z_ref, fcw_ref,
                  fcb_ref, wl_ref, wr_ref, x1_ref, y_ref, z2_ref):
    x1 = _mean_norm(s0_ref[...] + s1_ref[...], d0_ref[...] + d1_ref[...],
                    b_ref[...], z_ref[...])
    x1_ref[...] = x1
    xb = (jnp.dot(x1, fcw_ref[...], preferred_element_type=jnp.float32)
          + fcb_ref[...])
    y_ref[...] = jnp.dot(xb, wl_ref[...], preferred_element_type=jnp.float32)
    z2_ref[...] = jnp.dot(xb, wr_ref[...], preferred_element_type=jnp.float32)


def _comb_h2_body(s0_ref, s1_ref, d0_ref, d1_ref, b_ref, z_ref, wr_ref,
                  h2_ref, z2_ref):
    h2 = _mean_norm(s0_ref[...] + s1_ref[...], d0_ref[...] + d1_ref[...],
                    b_ref[...], z_ref[...])
    h2_ref[...] = h2
    z2_ref[...] = jnp.dot(h2, wr_ref[...], preferred_element_type=jnp.float32)


def _final_body(s0_ref, s1_ref, d0_ref, d1_ref, b_ref, z_ref, wl_ref,
                out_ref):
    mean_src = s0_ref[...] + s1_ref[...]
    dg = jnp.sum((d0_ref[...] + d1_ref[...])[:, D_OUT:], axis=1,
                 keepdims=True) * (1.0 / D_OUT)
    dg = jnp.maximum(dg, 1.0)
    mean = mean_src / dg
    pre = (jnp.dot(mean, wl_ref[...], preferred_element_type=jnp.float32)
           + b_ref[...] + z_ref[...])
    nrm = jnp.sqrt(jnp.sum(pre * pre, axis=1, keepdims=True))
    out_ref[...] = pre / jnp.maximum(nrm, 1e-12)


def _grid_call(body, in_specs, out_specs, out_shapes):
    return pl.pallas_call(
        body,
        grid=(NROW // BR,),
        in_specs=in_specs,
        out_specs=out_specs,
        out_shape=out_shapes,
    )


def kernel(features1, edge_index1,
           enc1_Wl, enc1_bl, enc1_Wr,
           enc2_Wl, enc2_bl, enc2_Wr,
           fc_W, fc_b,
           dec1_Wl, dec1_bl, dec1_Wr,
           dec2_Wl, dec2_bl, dec2_Wr):
    f32 = jnp.float32
    src = edge_index1[0].astype(jnp.int32)
    dst = edge_index1[1].astype(jnp.int32)
    pad = E_PAD - E
    pad_src = jnp.arange(pad, dtype=jnp.int32) % N
    pad_dst = N + (jnp.arange(pad, dtype=jnp.int32) % (NROW - N))
    src_p = jnp.concatenate([src, pad_src])
    dst_p = jnp.concatenate([dst, pad_dst])
    srcA = src_p.reshape(E_PAD // CH1, CH1)
    dstA = dst_p.reshape(E_PAD // CH1, CH1)
    srcB = src_p.reshape(E_PAD // CH2, CH2)
    dstB = dst_p.reshape(E_PAD // CH2, CH2)
    x0 = jnp.concatenate([features1, jnp.zeros((NROW - N, D_IN), f32)])

    w1l = enc1_Wl.T.astype(f32)
    w1r = enc1_Wr.T.astype(f32)
    w2l = enc2_Wl.T.astype(f32)
    w2r = enc2_Wr.T.astype(f32)
    fcw = fc_W.T.astype(f32)
    w3l = dec1_Wl.T.astype(f32)
    w3r = dec1_Wr.T.astype(f32)
    w4l = dec2_Wl.T.astype(f32)
    w4r = dec2_Wr.T.astype(f32)
    b1 = enc1_bl.reshape(1, D_OUT)
    b2 = enc2_bl.reshape(1, D_OUT)
    fcb = fc_b.reshape(1, D_OUT)
    b3 = dec1_bl.reshape(1, D_OUT)
    b4 = dec2_bl.reshape(1, D_IN)

    y1, z1 = _grid_call(
        _prep_body,
        [_row_spec(D_IN), _full_spec(D_IN, D_OUT), _full_spec(D_IN, D_OUT)],
        [_row_spec(D_IN), _row_spec(D_OUT)],
        [jax.ShapeDtypeStruct((NROW, D_IN), f32),
         jax.ShapeDtypeStruct((NROW, D_OUT), f32)],
    )(x0, w1l, w1r)

    sp = _segsum128(y1, srcA, dstA)[0]
    d0, d1 = sp[:NROW], sp[NROW:]

    y2, z2 = _grid_call(
        _comb2_body,
        [_row_spec(D_IN), _row_spec(D_IN), _full_spec(1, D_OUT),
         _row_spec(D_OUT), _full_spec(D_OUT, D_OUT),
         _full_spec(D_OUT, D_OUT)],
        [_row_spec(D_OUT), _row_spec(D_OUT)],
        [jax.ShapeDtypeStruct((NROW, D_OUT), f32)] * 2,
    )(d0, d1, b1, z1, w2l, w2r)

    sp2 = _segsum64(y2, srcB, dstB)[0]

    x1f, y3, z3 = _grid_call(
        _comb_fc_body,
        [_row_spec(D_OUT), _row_spec(D_OUT), _row_spec(D_IN),
         _row_spec(D_IN), _full_spec(1, D_OUT), _row_spec(D_OUT),
         _full_spec(D_OUT, D_OUT), _full_spec(1, D_OUT),
         _full_spec(D_OUT, D_OUT), _full_spec(D_OUT, D_OUT)],
        [_row_spec(D_OUT)] * 3,
        [jax.ShapeDtypeStruct((NROW, D_OUT), f32)] * 3,
    )(sp2[:NROW], sp2[NROW:], d0, d1, b2, z2, fcw, fcb, w3l, w3r)

    sp3 = _segsum64(y3, srcB, dstB)[0]

    h2, z4 = _grid_call(
        _comb_h2_body,
        [_row_spec(D_OUT), _row_spec(D_OUT), _row_spec(D_IN),
         _row_spec(D_IN), _full_spec(1, D_OUT), _row_spec(D_OUT),
         _full_spec(D_OUT, D_IN)],
        [_row_spec(D_OUT), _row_spec(D_IN)],
        [jax.ShapeDtypeStruct((NROW, D_OUT), f32),
         jax.ShapeDtypeStruct((NROW, D_IN), f32)],
    )(sp3[:NROW], sp3[NROW:], d0, d1, b3, z3, w4r)

    sp4 = _segsum64(h2, srcB, dstB)[0]

    x1_rec = _grid_call(
        _final_body,
        [_row_spec(D_OUT), _row_spec(D_OUT), _row_spec(D_IN),
         _row_spec(D_IN), _full_spec(1, D_IN), _row_spec(D_IN),
         _full_spec(D_OUT, D_IN)],
        _row_spec(D_IN),
        jax.ShapeDtypeStruct((NROW, D_IN), f32),
    )(sp4[:NROW], sp4[NROW:], d0, d1, b4, z4, w4l)

    return (x1f[:N], x1_rec[:N])


# speedup vs baseline: 4.7395x; 1.2142x over previous
import jax
import jax.numpy as jnp
from jax import lax
from jax.experimental import pallas as pl
from jax.experimental.pallas import tpu as pltpu
from jax.experimental.pallas import tpu_sc as plsc

N = 10000
E = 320000
D_IN = 128
D_OUT = 64

NC = 2
NS = 16
NW = NC * NS
E_PAD = 327680
EPW = E_PAD // NW
STRIPE = 640
NROW = NS * STRIPE
SROWS = 40


_SC_MESH = plsc.VectorSubcoreMesh(core_axis_name="c", subcore_axis_name="s")


def _make_segsum(w, ch, rb, nhalf, name):
    jpw = EPW // ch
    jph = jpw // nhalf

    def body(y_hbm, src_hbm, dst_hbm, out_hbm, srcv, dstv, rows, acc, gsem):
        c = lax.axis_index("c")
        s = lax.axis_index("s")
        wid = s * NC + c

        def zrow(r, _):
            for q in range(w // 16):
                rows[r, pl.ds(q * 16, 16)] = jnp.zeros((16,), jnp.float32)
            return 0

        lax.fori_loop(0, SROWS, zrow, 0)

        def zchunk(k, _):
            pltpu.sync_copy(rows.at[pl.ds(0, SROWS)],
                            acc.at[pl.ds(s * STRIPE + k * SROWS, SROWS)])
            return 0

        lax.fori_loop(0, STRIPE // SROWS, zchunk, 0)
        plsc.subcore_barrier()

        def gslot(j):
            return pl.ds(lax.rem(j, rb) * ch, ch)

        def half(h, _):
            base = wid * jpw + h * jph
            pltpu.sync_copy(src_hbm.at[pl.ds(base, jph)], srcv)
            pltpu.sync_copy(dst_hbm.at[pl.ds(base, jph)], dstv)
            for j in range(rb - 1):
                pltpu.async_copy(y_hbm.at[srcv.at[j]], rows.at[gslot(j)],
                                 gsem)

            def ibody(j, _):
                pltpu.make_async_copy(y_hbm.at[srcv.at[j]],
                                      rows.at[gslot(j)], gsem).wait()
                jn = lax.min(j + rb - 1, jph - 1)

                @pl.when(j + rb - 1 < jph)
                def _():
                    pltpu.async_copy(y_hbm.at[srcv.at[jn]],
                                     rows.at[gslot(jn)], gsem)

                pltpu.sync_copy(rows.at[gslot(j)], acc.at[dstv.at[j]],
                                add=True)
                return 0

            lax.fori_loop(0, jph, ibody, 0)
            return 0

        lax.fori_loop(0, nhalf, half, 0)
        plsc.subcore_barrier()

        def ochunk(k, _):
            r0 = s * STRIPE + k * SROWS
            pltpu.sync_copy(acc.at[pl.ds(r0, SROWS)],
                            rows.at[pl.ds(0, SROWS)])
            pltpu.sync_copy(rows.at[pl.ds(0, SROWS)],
                            out_hbm.at[pl.ds(c * NROW + r0, SROWS)])
            return 0

        lax.fori_loop(0, STRIPE // SROWS, ochunk, 0)

    return pl.kernel(
        body,
        out_type=[jax.ShapeDtypeStruct((2 * NROW, w), jnp.float32)],
        mesh=_SC_MESH,
        scratch_types=[
            pltpu.VMEM((jph, ch), jnp.int32),
            pltpu.VMEM((jph, ch), jnp.int32),
            pltpu.VMEM((rb * ch, w), jnp.float32),
            pltpu.VMEM_SHARED((NROW, w), jnp.float32),
            pltpu.SemaphoreType.DMA,
        ],
        compiler_params=pltpu.CompilerParams(use_tc_tiling_on_sc=False),
        name=name,
    )


CH2 = 128
_segsum64 = _make_segsum(D_OUT, CH2, 3, 2, "segsum64")

DWD = 16
_DEG_JPW = EPW // CH2
_DEG_K = 8


def _deg_body(dst_hbm, deg_hbm, dstv, ones, dstage, dacc, ssem):
    c = lax.axis_index("c")
    s = lax.axis_index("s")
    wid = s * NC + c

    def orow(r, _):
        ones[r, pl.ds(0, 16)] = jnp.ones((16,), jnp.float32)
        return 0

    lax.fori_loop(0, CH2, orow, 0)

    def zrow(r, _):
        dstage[r, pl.ds(0, 16)] = jnp.zeros((16,), jnp.float32)
        return 0

    lax.fori_loop(0, SROWS, zrow, 0)

    def zchunk(k, _):
        pltpu.sync_copy(dstage,
                        dacc.at[pl.ds(s * STRIPE + k * SROWS, SROWS)])
        return 0

    lax.fori_loop(0, STRIPE // SROWS, zchunk, 0)

    pltpu.sync_copy(dst_hbm.at[pl.ds(wid * _DEG_JPW, _DEG_JPW)], dstv)
    plsc.subcore_barrier()

    def group(g, _):
        for i in range(_DEG_K):
            pltpu.async_copy(ones, dacc.at[dstv.at[g * _DEG_K + i]], ssem,
                             add=True)
        for i in range(_DEG_K):
            pltpu.make_async_copy(ones, dacc.at[dstv.at[g * _DEG_K + i]],
                                  ssem).wait()
        return 0

    lax.fori_loop(0, _DEG_JPW // _DEG_K, group, 0)
    plsc.subcore_barrier()

    def ochunk(k, _):
        r0 = s * STRIPE + k * SROWS
        pltpu.sync_copy(dacc.at[pl.ds(r0, SROWS)], dstage)
        pltpu.sync_copy(dstage, deg_hbm.at[pl.ds(c * NROW + r0, SROWS)])
        return 0

    lax.fori_loop(0, STRIPE // SROWS, ochunk, 0)


_degscatter = pl.kernel(
    _deg_body,
    out_type=[jax.ShapeDtypeStruct((2 * NROW, DWD), jnp.float32)],
    mesh=_SC_MESH,
    scratch_types=[
        pltpu.VMEM((_DEG_JPW, CH2), jnp.int32),
        pltpu.VMEM((CH2, DWD), jnp.float32),
        pltpu.VMEM((SROWS, DWD), jnp.float32),
        pltpu.VMEM_SHARED((NROW, DWD), jnp.float32),
        pltpu.SemaphoreType.DMA,
    ],
    compiler_params=pltpu.CompilerParams(use_tc_tiling_on_sc=False),
    name="degscatter",
)


BR = 640


def _row_spec(w):
    return pl.BlockSpec((BR, w), lambda i: (i, 0))


def _full_spec(a, b):
    return pl.BlockSpec((a, b), lambda i: (0, 0))


def _prep_body(x_ref, wl_ref, wr_ref, y_ref, z_ref):
    x = x_ref[...]
    y_ref[...] = jnp.dot(x, wl_ref[...], preferred_element_type=jnp.float32)
    z_ref[...] = jnp.dot(x, wr_ref[...], preferred_element_type=jnp.float32)


def _mean_norm(sw, d, b, z):
    s = sw[:, :D_OUT]
    dg = jnp.sum(d, axis=1, keepdims=True) * (1.0 / DWD)
    dg = jnp.maximum(dg, 1.0)
    pre = s / dg + b + z
    nrm = jnp.sqrt(jnp.sum(pre * pre, axis=1, keepdims=True))
    return pre / jnp.maximum(nrm, 1e-12)


def _comb2_body(s0_ref, s1_ref, d0_ref, d1_ref, b_ref, z_ref, wl_ref,
                wr_ref, y_ref, z2_ref):
    h = _mean_norm(s0_ref[...] + s1_ref[...], d0_ref[...] + d1_ref[...],
                   b_ref[...], z_ref[...])
    y_ref[...] = jnp.dot(h, wl_ref[...], preferred_element_type=jnp.float32)
    z2_ref[...] = jnp.dot(h, wr_ref[...], preferred_element_type=jnp.float32)


def _comb_fc_body(s0_ref, s1_ref, d0_ref, d1_ref, b_ref, z_ref, fcw_ref,
                  fcb_ref, wl_ref, wr_ref, x1_ref, y_ref, z2_ref):
    x1 = _mean_norm(s0_ref[...] + s1_ref[...], d0_ref[...] + d1_ref[...],
                    b_ref[...], z_ref[...])
    x1_ref[...] = x1
    xb = (jnp.dot(x1, fcw_ref[...], preferred_element_type=jnp.float32)
          + fcb_ref[...])
    y_ref[...] = jnp.dot(xb, wl_ref[...], preferred_element_type=jnp.float32)
    z2_ref[...] = jnp.dot(xb, wr_ref[...], preferred_element_type=jnp.float32)


def _comb_h2_body(s0_ref, s1_ref, d0_ref, d1_ref, b_ref, z_ref, wr_ref,
                  h2_ref, z2_ref):
    h2 = _mean_norm(s0_ref[...] + s1_ref[...], d0_ref[...] + d1_ref[...],
                    b_ref[...], z_ref[...])
    h2_ref[...] = h2
    z2_ref[...] = jnp.dot(h2, wr_ref[...], preferred_element_type=jnp.float32)


def _final_body(s0_ref, s1_ref, d0_ref, d1_ref, b_ref, z_ref, wl_ref,
                out_ref):
    mean_src = s0_ref[...] + s1_ref[...]
    dg = jnp.sum(d0_ref[...] + d1_ref[...], axis=1,
                 keepdims=True) * (1.0 / DWD)
    dg = jnp.maximum(dg, 1.0)
    mean = mean_src / dg
    pre = (jnp.dot(mean, wl_ref[...], preferred_element_type=jnp.float32)
           + b_ref[...] + z_ref[...])
    nrm = jnp.sqrt(jnp.sum(pre * pre, axis=1, keepdims=True))
    out_ref[...] = pre / jnp.maximum(nrm, 1e-12)


def _grid_call(body, in_specs, out_specs, out_shapes):
    return pl.pallas_call(
        body,
        grid=(NROW // BR,),
        in_specs=in_specs,
        out_specs=out_specs,
        out_shape=out_shapes,
    )


def kernel(features1, edge_index1,
           enc1_Wl, enc1_bl, enc1_Wr,
           enc2_Wl, enc2_bl, enc2_Wr,
           fc_W, fc_b,
           dec1_Wl, dec1_bl, dec1_Wr,
           dec2_Wl, dec2_bl, dec2_Wr):
    f32 = jnp.float32
    src = edge_index1[0].astype(jnp.int32)
    dst = edge_index1[1].astype(jnp.int32)
    pad = E_PAD - E
    pad_src = jnp.arange(pad, dtype=jnp.int32) % N
    pad_dst = N + (jnp.arange(pad, dtype=jnp.int32) % (NROW - N))
    src_p = jnp.concatenate([src, pad_src])
    dst_p = jnp.concatenate([dst, pad_dst])
    srcB = src_p.reshape(E_PAD // CH2, CH2)
    dstB = dst_p.reshape(E_PAD // CH2, CH2)
    x0 = jnp.concatenate([features1, jnp.zeros((NROW - N, D_IN), f32)])

    w1l = enc1_Wl.T.astype(f32)
    w1r = enc1_Wr.T.astype(f32)
    w2l = enc2_Wl.T.astype(f32)
    w2r = enc2_Wr.T.astype(f32)
    fcw = fc_W.T.astype(f32)
    w3l = dec1_Wl.T.astype(f32)
    w3r = dec1_Wr.T.astype(f32)
    w4l = dec2_Wl.T.astype(f32)
    w4r = dec2_Wr.T.astype(f32)
    b1 = enc1_bl.reshape(1, D_OUT)
    b2 = enc2_bl.reshape(1, D_OUT)
    fcb = fc_b.reshape(1, D_OUT)
    b3 = dec1_bl.reshape(1, D_OUT)
    b4 = dec2_bl.reshape(1, D_IN)

    degp = _degscatter(dstB)[0]
    d0, d1 = degp[:NROW], degp[NROW:]

    y1, z1 = _grid_call(
        _prep_body,
        [_row_spec(D_IN), _full_spec(D_IN, D_OUT), _full_spec(D_IN, D_OUT)],
        [_row_spec(D_OUT), _row_spec(D_OUT)],
        [jax.ShapeDtypeStruct((NROW, D_OUT), f32)] * 2,
    )(x0, w1l, w1r)

    sp = _segsum64(y1, srcB, dstB)[0]

    y2, z2 = _grid_call(
        _comb2_body,
        [_row_spec(D_OUT), _row_spec(D_OUT), _row_spec(DWD),
         _row_spec(DWD), _full_spec(1, D_OUT),
         _row_spec(D_OUT), _full_spec(D_OUT, D_OUT),
         _full_spec(D_OUT, D_OUT)],
        [_row_spec(D_OUT), _row_spec(D_OUT)],
        [jax.ShapeDtypeStruct((NROW, D_OUT), f32)] * 2,
    )(sp[:NROW], sp[NROW:], d0, d1, b1, z1, w2l, w2r)

    sp2 = _segsum64(y2, srcB, dstB)[0]

    x1f, y3, z3 = _grid_call(
        _comb_fc_body,
        [_row_spec(D_OUT), _row_spec(D_OUT), _row_spec(DWD),
         _row_spec(DWD), _full_spec(1, D_OUT), _row_spec(D_OUT),
         _full_spec(D_OUT, D_OUT), _full_spec(1, D_OUT),
         _full_spec(D_OUT, D_OUT), _full_spec(D_OUT, D_OUT)],
        [_row_spec(D_OUT)] * 3,
        [jax.ShapeDtypeStruct((NROW, D_OUT), f32)] * 3,
    )(sp2[:NROW], sp2[NROW:], d0, d1, b2, z2, fcw, fcb, w3l, w3r)

    sp3 = _segsum64(y3, srcB, dstB)[0]

    h2, z4 = _grid_call(
        _comb_h2_body,
        [_row_spec(D_OUT), _row_spec(D_OUT), _row_spec(DWD),
         _row_spec(DWD), _full_spec(1, D_OUT), _row_spec(D_OUT),
         _full_spec(D_OUT, D_IN)],
        [_row_spec(D_OUT), _row_spec(D_IN)],
        [jax.ShapeDtypeStruct((NROW, D_OUT), f32),
         jax.ShapeDtypeStruct((NROW, D_IN), f32)],
    )(sp3[:NROW], sp3[NROW:], d0, d1, b3, z3, w4r)

    sp4 = _segsum64(h2, srcB, dstB)[0]

    x1_rec = _grid_call(
        _final_body,
        [_row_spec(D_OUT), _row_spec(D_OUT), _row_spec(DWD),
         _row_spec(DWD), _full_spec(1, D_IN), _row_spec(D_IN),
         _full_spec(D_OUT, D_IN)],
        _row_spec(D_IN),
        jax.ShapeDtypeStruct((NROW, D_IN), f32),
    )(sp4[:NROW], sp4[NROW:], d0, d1, b4, z4, w4l)

    return (x1f[:N], x1_rec[:N])


# speedup vs baseline: 4.9645x; 1.0475x over previous
import jax
import jax.numpy as jnp
from jax import lax
from jax.experimental import pallas as pl
from jax.experimental.pallas import tpu as pltpu
from jax.experimental.pallas import tpu_sc as plsc

N = 10000
E = 320000
D_IN = 128
D_OUT = 64

NC = 2
NS = 16
NW = NC * NS
E_PAD = 327680
EPW = E_PAD // NW
STRIPE = 640
NROW = NS * STRIPE
SROWS = 40


_SC_MESH = plsc.VectorSubcoreMesh(core_axis_name="c", subcore_axis_name="s")


def _make_segsum(w, ch, rb, nhalf, name):
    jpw = EPW // ch
    jph = jpw // nhalf

    def body(y_hbm, src_hbm, dst_hbm, out_hbm, srcv, dstv, rows, acc, gsem):
        c = lax.axis_index("c")
        s = lax.axis_index("s")
        wid = s * NC + c

        def zrow(r, _):
            for q in range(w // 16):
                rows[r, pl.ds(q * 16, 16)] = jnp.zeros((16,), jnp.float32)
            return 0

        lax.fori_loop(0, SROWS, zrow, 0)

        def zchunk(k, _):
            pltpu.sync_copy(rows.at[pl.ds(0, SROWS)],
                            acc.at[pl.ds(s * STRIPE + k * SROWS, SROWS)])
            return 0

        lax.fori_loop(0, STRIPE // SROWS, zchunk, 0)
        plsc.subcore_barrier()

        def gslot(j):
            return pl.ds(lax.rem(j, rb) * ch, ch)

        def half(h, _):
            base = wid * jpw + h * jph
            pltpu.sync_copy(src_hbm.at[pl.ds(base, jph)], srcv)
            pltpu.sync_copy(dst_hbm.at[pl.ds(base, jph)], dstv)
            for j in range(rb - 1):
                pltpu.async_copy(y_hbm.at[srcv.at[j]], rows.at[gslot(j)],
                                 gsem)

            def ibody(j, _):
                pltpu.make_async_copy(y_hbm.at[srcv.at[j]],
                                      rows.at[gslot(j)], gsem).wait()
                jn = lax.min(j + rb - 1, jph - 1)

                @pl.when(j + rb - 1 < jph)
                def _():
                    pltpu.async_copy(y_hbm.at[srcv.at[jn]],
                                     rows.at[gslot(jn)], gsem)

                pltpu.sync_copy(rows.at[gslot(j)], acc.at[dstv.at[j]],
                                add=True)
                return 0

            lax.fori_loop(0, jph, ibody, 0)
            return 0

        lax.fori_loop(0, nhalf, half, 0)
        plsc.subcore_barrier()

        def ochunk(k, _):
            r0 = s * STRIPE + k * SROWS
            pltpu.sync_copy(acc.at[pl.ds(r0, SROWS)],
                            rows.at[pl.ds(0, SROWS)])
            pltpu.sync_copy(rows.at[pl.ds(0, SROWS)],
                            out_hbm.at[pl.ds(c * NROW + r0, SROWS)])
            return 0

        lax.fori_loop(0, STRIPE // SROWS, ochunk, 0)

    return pl.kernel(
        body,
        out_type=[jax.ShapeDtypeStruct((2 * NROW, w), jnp.float32)],
        mesh=_SC_MESH,
        scratch_types=[
            pltpu.VMEM((jph, ch), jnp.int32),
            pltpu.VMEM((jph, ch), jnp.int32),
            pltpu.VMEM((rb * ch, w), jnp.float32),
            pltpu.VMEM_SHARED((NROW, w), jnp.float32),
            pltpu.SemaphoreType.DMA,
        ],
        compiler_params=pltpu.CompilerParams(use_tc_tiling_on_sc=False),
        name=name,
    )


CH2 = 128
_segsum64 = _make_segsum(D_OUT, CH2, 3, 2, "segsum64")

DWD = 16
_DEG_JPW = EPW // CH2
_DEG_K = 8


def _deg_body(dst_hbm, deg_hbm, dstv, ones, dstage, dacc, ssem):
    c = lax.axis_index("c")
    s = lax.axis_index("s")
    wid = s * NC + c

    def orow(r, _):
        ones[r, pl.ds(0, 16)] = jnp.ones((16,), jnp.float32)
        return 0

    lax.fori_loop(0, CH2, orow, 0)

    def zrow(r, _):
        dstage[r, pl.ds(0, 16)] = jnp.zeros((16,), jnp.float32)
        return 0

    lax.fori_loop(0, SROWS, zrow, 0)

    def zchunk(k, _):
        pltpu.sync_copy(dstage,
                        dacc.at[pl.ds(s * STRIPE + k * SROWS, SROWS)])
        return 0

    lax.fori_loop(0, STRIPE // SROWS, zchunk, 0)

    pltpu.sync_copy(dst_hbm.at[pl.ds(wid * _DEG_JPW, _DEG_JPW)], dstv)
    plsc.subcore_barrier()

    def group(g, _):
        for i in range(_DEG_K):
            pltpu.async_copy(ones, dacc.at[dstv.at[g * _DEG_K + i]], ssem,
                             add=True)
        for i in range(_DEG_K):
            pltpu.make_async_copy(ones, dacc.at[dstv.at[g * _DEG_K + i]],
                                  ssem).wait()
        return 0

    lax.fori_loop(0, _DEG_JPW // _DEG_K, group, 0)
    plsc.subcore_barrier()

    def ochunk(k, _):
        r0 = s * STRIPE + k * SROWS
        pltpu.sync_copy(dacc.at[pl.ds(r0, SROWS)], dstage)
        pltpu.sync_copy(dstage, deg_hbm.at[pl.ds(c * NROW + r0, SROWS)])
        return 0

    lax.fori_loop(0, STRIPE // SROWS, ochunk, 0)


_degscatter = pl.kernel(
    _deg_body,
    out_type=[jax.ShapeDtypeStruct((2 * NROW, DWD), jnp.float32)],
    mesh=_SC_MESH,
    scratch_types=[
        pltpu.VMEM((_DEG_JPW, CH2), jnp.int32),
        pltpu.VMEM((CH2, DWD), jnp.float32),
        pltpu.VMEM((SROWS, DWD), jnp.float32),
        pltpu.VMEM_SHARED((NROW, DWD), jnp.float32),
        pltpu.SemaphoreType.DMA,
    ],
    compiler_params=pltpu.CompilerParams(use_tc_tiling_on_sc=False),
    name="degscatter",
)


BR = 2560


def _row_spec(w):
    return pl.BlockSpec((BR, w), lambda i: (i, 0))


def _full_spec(a, b):
    return pl.BlockSpec((a, b), lambda i: (0, 0))


def _prep_body(x_ref, wl_ref, wr_ref, y_ref, z_ref):
    x = x_ref[...]
    y_ref[...] = jnp.dot(x, wl_ref[...], preferred_element_type=jnp.float32)
    z_ref[...] = jnp.dot(x, wr_ref[...], preferred_element_type=jnp.float32)


def _mean_norm(sw, d, b, z):
    s = sw[:, :D_OUT]
    dg = jnp.sum(d, axis=1, keepdims=True) * (1.0 / DWD)
    dg = jnp.maximum(dg, 1.0)
    pre = s / dg + b + z
    nrm = jnp.sqrt(jnp.sum(pre * pre, axis=1, keepdims=True))
    return pre / jnp.maximum(nrm, 1e-12)


def _comb2_y(s0_ref, s1_ref, d0_ref, d1_ref, b_ref, z_ref, wl_ref, y_ref):
    h = _mean_norm(s0_ref[...] + s1_ref[...], d0_ref[...] + d1_ref[...],
                   b_ref[...], z_ref[...])
    y_ref[...] = jnp.dot(h, wl_ref[...], preferred_element_type=jnp.float32)


def _comb2_z(s0_ref, s1_ref, d0_ref, d1_ref, b_ref, z_ref, wr_ref, z2_ref):
    h = _mean_norm(s0_ref[...] + s1_ref[...], d0_ref[...] + d1_ref[...],
                   b_ref[...], z_ref[...])
    z2_ref[...] = jnp.dot(h, wr_ref[...], preferred_element_type=jnp.float32)


def _comb_fc_y(s0_ref, s1_ref, d0_ref, d1_ref, b_ref, z_ref, fcw_ref,
               fcb_ref, wl_ref, y_ref):
    x1 = _mean_norm(s0_ref[...] + s1_ref[...], d0_ref[...] + d1_ref[...],
                    b_ref[...], z_ref[...])
    xb = (jnp.dot(x1, fcw_ref[...], preferred_element_type=jnp.float32)
          + fcb_ref[...])
    y_ref[...] = jnp.dot(xb, wl_ref[...], preferred_element_type=jnp.float32)


def _comb_fc_rest(s0_ref, s1_ref, d0_ref, d1_ref, b_ref, z_ref, fcw_ref,
                  fcb_ref, wr_ref, x1_ref, z2_ref):
    x1 = _mean_norm(s0_ref[...] + s1_ref[...], d0_ref[...] + d1_ref[...],
                    b_ref[...], z_ref[...])
    x1_ref[...] = x1
    xb = (jnp.dot(x1, fcw_ref[...], preferred_element_type=jnp.float32)
          + fcb_ref[...])
    z2_ref[...] = jnp.dot(xb, wr_ref[...], preferred_element_type=jnp.float32)


def _comb_h2_h(s0_ref, s1_ref, d0_ref, d1_ref, b_ref, z_ref, h2_ref):
    h2_ref[...] = _mean_norm(s0_ref[...] + s1_ref[...],
                             d0_ref[...] + d1_ref[...],
                             b_ref[...], z_ref[...])


def _matmul_z(h_ref, wr_ref, z_ref):
    z_ref[...] = jnp.dot(h_ref[...], wr_ref[...],
                         preferred_element_type=jnp.float32)


def _final_body(s0_ref, s1_ref, d0_ref, d1_ref, b_ref, z_ref, wl_ref,
                out_ref):
    mean_src = s0_ref[...] + s1_ref[...]
    dg = jnp.sum(d0_ref[...] + d1_ref[...], axis=1,
                 keepdims=True) * (1.0 / DWD)
    dg = jnp.maximum(dg, 1.0)
    mean = mean_src / dg
    pre = (jnp.dot(mean, wl_ref[...], preferred_element_type=jnp.float32)
           + b_ref[...] + z_ref[...])
    nrm = jnp.sqrt(jnp.sum(pre * pre, axis=1, keepdims=True))
    out_ref[...] = pre / jnp.maximum(nrm, 1e-12)


def _grid_call(body, in_specs, out_specs, out_shapes):
    return pl.pallas_call(
        body,
        grid=(NROW // BR,),
        in_specs=in_specs,
        out_specs=out_specs,
        out_shape=out_shapes,
    )


def kernel(features1, edge_index1,
           enc1_Wl, enc1_bl, enc1_Wr,
           enc2_Wl, enc2_bl, enc2_Wr,
           fc_W, fc_b,
           dec1_Wl, dec1_bl, dec1_Wr,
           dec2_Wl, dec2_bl, dec2_Wr):
    f32 = jnp.float32
    src = edge_index1[0].astype(jnp.int32)
    dst = edge_index1[1].astype(jnp.int32)
    pad = E_PAD - E
    pad_src = jnp.arange(pad, dtype=jnp.int32) % N
    pad_dst = N + (jnp.arange(pad, dtype=jnp.int32) % (NROW - N))
    src_p = jnp.concatenate([src, pad_src])
    dst_p = jnp.concatenate([dst, pad_dst])
    srcB = src_p.reshape(E_PAD // CH2, CH2)
    dstB = dst_p.reshape(E_PAD // CH2, CH2)
    x0 = jnp.concatenate([features1, jnp.zeros((NROW - N, D_IN), f32)])

    w1l = enc1_Wl.T.astype(f32)
    w1r = enc1_Wr.T.astype(f32)
    w2l = enc2_Wl.T.astype(f32)
    w2r = enc2_Wr.T.astype(f32)
    fcw = fc_W.T.astype(f32)
    w3l = dec1_Wl.T.astype(f32)
    w3r = dec1_Wr.T.astype(f32)
    w4l = dec2_Wl.T.astype(f32)
    w4r = dec2_Wr.T.astype(f32)
    b1 = enc1_bl.reshape(1, D_OUT)
    b2 = enc2_bl.reshape(1, D_OUT)
    fcb = fc_b.reshape(1, D_OUT)
    b3 = dec1_bl.reshape(1, D_OUT)
    b4 = dec2_bl.reshape(1, D_IN)

    degp = _degscatter(dstB)[0]
    d0, d1 = degp[:NROW], degp[NROW:]

    y1, z1 = _grid_call(
        _prep_body,
        [_row_spec(D_IN), _full_spec(D_IN, D_OUT), _full_spec(D_IN, D_OUT)],
        [_row_spec(D_OUT), _row_spec(D_OUT)],
        [jax.ShapeDtypeStruct((NROW, D_OUT), f32)] * 2,
    )(x0, w1l, w1r)

    sp = _segsum64(y1, srcB, dstB)[0]

    comb_in = [_row_spec(D_OUT), _row_spec(D_OUT), _row_spec(DWD),
               _row_spec(DWD), _full_spec(1, D_OUT), _row_spec(D_OUT)]
    y2 = _grid_call(
        _comb2_y,
        comb_in + [_full_spec(D_OUT, D_OUT)],
        _row_spec(D_OUT),
        jax.ShapeDtypeStruct((NROW, D_OUT), f32),
    )(sp[:NROW], sp[NROW:], d0, d1, b1, z1, w2l)
    z2 = _grid_call(
        _comb2_z,
        comb_in + [_full_spec(D_OUT, D_OUT)],
        _row_spec(D_OUT),
        jax.ShapeDtypeStruct((NROW, D_OUT), f32),
    )(sp[:NROW], sp[NROW:], d0, d1, b1, z1, w2r)

    sp2 = _segsum64(y2, srcB, dstB)[0]

    y3 = _grid_call(
        _comb_fc_y,
        comb_in + [_full_spec(D_OUT, D_OUT), _full_spec(1, D_OUT),
                   _full_spec(D_OUT, D_OUT)],
        _row_spec(D_OUT),
        jax.ShapeDtypeStruct((NROW, D_OUT), f32),
    )(sp2[:NROW], sp2[NROW:], d0, d1, b2, z2, fcw, fcb, w3l)
    x1f, z3 = _grid_call(
        _comb_fc_rest,
        comb_in + [_full_spec(D_OUT, D_OUT), _full_spec(1, D_OUT),
                   _full_spec(D_OUT, D_OUT)],
        [_row_spec(D_OUT), _row_spec(D_OUT)],
        [jax.ShapeDtypeStruct((NROW, D_OUT), f32)] * 2,
    )(sp2[:NROW], sp2[NROW:], d0, d1, b2, z2, fcw, fcb, w3r)

    sp3 = _segsum64(y3, srcB, dstB)[0]

    h2 = _grid_call(
        _comb_h2_h,
        comb_in,
        _row_spec(D_OUT),
        jax.ShapeDtypeStruct((NROW, D_OUT), f32),
    )(sp3[:NROW], sp3[NROW:], d0, d1, b3, z3)
    z4 = _grid_call(
        _matmul_z,
        [_row_spec(D_OUT), _full_spec(D_OUT, D_IN)],
        _row_spec(D_IN),
        jax.ShapeDtypeStruct((NROW, D_IN), f32),
    )(h2, w4r)

    sp4 = _segsum64(h2, srcB, dstB)[0]

    x1_rec = _grid_call(
        _final_body,
        [_row_spec(D_OUT), _row_spec(D_OUT), _row_spec(DWD),
         _row_spec(DWD), _full_spec(1, D_IN), _row_spec(D_IN),
         _full_spec(D_OUT, D_IN)],
        _row_spec(D_IN),
        jax.ShapeDtypeStruct((NROW, D_IN), f32),
    )(sp4[:NROW], sp4[NROW:], d0, d1, b4, z4, w4l)

    return (x1f[:N], x1_rec[:N])


# speedup vs baseline: 5.1901x; 1.0454x over previous
import jax
import jax.numpy as jnp
from jax import lax
from jax.experimental import pallas as pl
from jax.experimental.pallas import tpu as pltpu
from jax.experimental.pallas import tpu_sc as plsc

N = 10000
E = 320000
D_IN = 128
D_OUT = 64

NC = 2
NS = 16
NW = NC * NS
E_PAD = 327680
EPW = E_PAD // NW
STRIPE = 640
NROW = NS * STRIPE
SROWS = 40


_SC_MESH = plsc.VectorSubcoreMesh(core_axis_name="c", subcore_axis_name="s")


def _make_segsum(w, ch, rb, nhalf, name):
    jpw = EPW // ch
    jph = jpw // nhalf

    def body(y_hbm, src_hbm, dst_hbm, out_hbm, srcv, dstv, rows, acc, gsem):
        c = lax.axis_index("c")
        s = lax.axis_index("s")
        wid = s * NC + c

        def zrow(r, _):
            for q in range(w // 16):
                rows[r, pl.ds(q * 16, 16)] = jnp.zeros((16,), jnp.float32)
            return 0

        lax.fori_loop(0, SROWS, zrow, 0)

        def zchunk(k, _):
            pltpu.sync_copy(rows.at[pl.ds(0, SROWS)],
                            acc.at[pl.ds(s * STRIPE + k * SROWS, SROWS)])
            return 0

        lax.fori_loop(0, STRIPE // SROWS, zchunk, 0)
        plsc.subcore_barrier()

        def gslot(j):
            return pl.ds(lax.rem(j, rb) * ch, ch)

        def half(h, _):
            base = wid * jpw + h * jph
            pltpu.sync_copy(src_hbm.at[pl.ds(base, jph)], srcv)
            pltpu.sync_copy(dst_hbm.at[pl.ds(base, jph)], dstv)
            for j in range(rb - 1):
                pltpu.async_copy(y_hbm.at[srcv.at[j]], rows.at[gslot(j)],
                                 gsem)

            def ibody(j, _):
                pltpu.make_async_copy(y_hbm.at[srcv.at[j]],
                                      rows.at[gslot(j)], gsem).wait()
                jn = lax.min(j + rb - 1, jph - 1)

                @pl.when(j + rb - 1 < jph)
                def _():
                    pltpu.async_copy(y_hbm.at[srcv.at[jn]],
                                     rows.at[gslot(jn)], gsem)

                pltpu.sync_copy(rows.at[gslot(j)], acc.at[dstv.at[j]],
                                add=True)
                return 0

            lax.fori_loop(0, jph, ibody, 0)
            return 0

        lax.fori_loop(0, nhalf, half, 0)
        plsc.subcore_barrier()

        def ochunk(k, _):
            r0 = s * STRIPE + k * SROWS
            pltpu.sync_copy(acc.at[pl.ds(r0, SROWS)],
                            rows.at[pl.ds(0, SROWS)])
            pltpu.sync_copy(rows.at[pl.ds(0, SROWS)],
                            out_hbm.at[pl.ds(c * NROW + r0, SROWS)])
            return 0

        lax.fori_loop(0, STRIPE // SROWS, ochunk, 0)

    return pl.kernel(
        body,
        out_type=[jax.ShapeDtypeStruct((2 * NROW, w), jnp.float32)],
        mesh=_SC_MESH,
        scratch_types=[
            pltpu.VMEM((jph, ch), jnp.int32),
            pltpu.VMEM((jph, ch), jnp.int32),
            pltpu.VMEM((rb * ch, w), jnp.float32),
            pltpu.VMEM_SHARED((NROW, w), jnp.float32),
            pltpu.SemaphoreType.DMA,
        ],
        compiler_params=pltpu.CompilerParams(use_tc_tiling_on_sc=False),
        name=name,
    )


CH2 = 128
_segsum64 = _make_segsum(D_OUT, CH2, 4, 2, "segsum64")

DWD = 16
_DEG_JPW = EPW // CH2
_DEG_K = 8


def _deg_body(dst_hbm, deg_hbm, dstv, ones, dstage, dacc, ssem):
    c = lax.axis_index("c")
    s = lax.axis_index("s")
    wid = s * NC + c

    def orow(r, _):
        ones[r, pl.ds(0, 16)] = jnp.ones((16,), jnp.float32)
        return 0

    lax.fori_loop(0, CH2, orow, 0)

    def zrow(r, _):
        dstage[r, pl.ds(0, 16)] = jnp.zeros((16,), jnp.float32)
        return 0

    lax.fori_loop(0, SROWS, zrow, 0)

    def zchunk(k, _):
        pltpu.sync_copy(dstage,
                        dacc.at[pl.ds(s * STRIPE + k * SROWS, SROWS)])
        return 0

    lax.fori_loop(0, STRIPE // SROWS, zchunk, 0)

    pltpu.sync_copy(dst_hbm.at[pl.ds(wid * _DEG_JPW, _DEG_JPW)], dstv)
    plsc.subcore_barrier()

    def group(g, _):
        for i in range(_DEG_K):
            pltpu.async_copy(ones, dacc.at[dstv.at[g * _DEG_K + i]], ssem,
                             add=True)
        for i in range(_DEG_K):
            pltpu.make_async_copy(ones, dacc.at[dstv.at[g * _DEG_K + i]],
                                  ssem).wait()
        return 0

    lax.fori_loop(0, _DEG_JPW // _DEG_K, group, 0)
    plsc.subcore_barrier()

    def ochunk(k, _):
        r0 = s * STRIPE + k * SROWS
        pltpu.sync_copy(dacc.at[pl.ds(r0, SROWS)], dstage)
        pltpu.sync_copy(dstage, deg_hbm.at[pl.ds(c * NROW + r0, SROWS)])
        return 0

    lax.fori_loop(0, STRIPE // SROWS, ochunk, 0)


_degscatter = pl.kernel(
    _deg_body,
    out_type=[jax.ShapeDtypeStruct((2 * NROW, DWD), jnp.float32)],
    mesh=_SC_MESH,
    scratch_types=[
        pltpu.VMEM((_DEG_JPW, CH2), jnp.int32),
        pltpu.VMEM((CH2, DWD), jnp.float32),
        pltpu.VMEM((SROWS, DWD), jnp.float32),
        pltpu.VMEM_SHARED((NROW, DWD), jnp.float32),
        pltpu.SemaphoreType.DMA,
    ],
    compiler_params=pltpu.CompilerParams(use_tc_tiling_on_sc=False),
    name="degscatter",
)


BR = 2560


def _row_spec(w):
    return pl.BlockSpec((BR, w), lambda i: (i, 0))


def _full_spec(a, b):
    return pl.BlockSpec((a, b), lambda i: (0, 0))


def _prep_body(x_ref, wl_ref, wr_ref, y_ref, z_ref):
    x = x_ref[...]
    y_ref[...] = jnp.dot(x, wl_ref[...], preferred_element_type=jnp.float32)
    z_ref[...] = jnp.dot(x, wr_ref[...], preferred_element_type=jnp.float32)


def _mean_norm(sw, d, b, z):
    s = sw[:, :D_OUT]
    dg = jnp.sum(d, axis=1, keepdims=True) * (1.0 / DWD)
    dg = jnp.maximum(dg, 1.0)
    pre = s / dg + b + z
    nrm = jnp.sqrt(jnp.sum(pre * pre, axis=1, keepdims=True))
    return pre / jnp.maximum(nrm, 1e-12)


def _comb2_y(s0_ref, s1_ref, d0_ref, d1_ref, b_ref, z_ref, wl_ref, y_ref):
    h = _mean_norm(s0_ref[...] + s1_ref[...], d0_ref[...] + d1_ref[...],
                   b_ref[...], z_ref[...])
    y_ref[...] = jnp.dot(h, wl_ref[...], preferred_element_type=jnp.float32)


def _comb2_z(s0_ref, s1_ref, d0_ref, d1_ref, b_ref, z_ref, wr_ref, z2_ref):
    h = _mean_norm(s0_ref[...] + s1_ref[...], d0_ref[...] + d1_ref[...],
                   b_ref[...], z_ref[...])
    z2_ref[...] = jnp.dot(h, wr_ref[...], preferred_element_type=jnp.float32)


def _comb_fc_y(s0_ref, s1_ref, d0_ref, d1_ref, b_ref, z_ref, fcw_ref,
               fcb_ref, wl_ref, y_ref):
    x1 = _mean_norm(s0_ref[...] + s1_ref[...], d0_ref[...] + d1_ref[...],
                    b_ref[...], z_ref[...])
    xb = (jnp.dot(x1, fcw_ref[...], preferred_element_type=jnp.float32)
          + fcb_ref[...])
    y_ref[...] = jnp.dot(xb, wl_ref[...], preferred_element_type=jnp.float32)


def _comb_fc_rest(s0_ref, s1_ref, d0_ref, d1_ref, b_ref, z_ref, fcw_ref,
                  fcb_ref, wr_ref, x1_ref, z2_ref):
    x1 = _mean_norm(s0_ref[...] + s1_ref[...], d0_ref[...] + d1_ref[...],
                    b_ref[...], z_ref[...])
    x1_ref[...] = x1
    xb = (jnp.dot(x1, fcw_ref[...], preferred_element_type=jnp.float32)
          + fcb_ref[...])
    z2_ref[...] = jnp.dot(xb, wr_ref[...], preferred_element_type=jnp.float32)


def _comb_h2_h(s0_ref, s1_ref, d0_ref, d1_ref, b_ref, z_ref, h2_ref):
    h2_ref[...] = _mean_norm(s0_ref[...] + s1_ref[...],
                             d0_ref[...] + d1_ref[...],
                             b_ref[...], z_ref[...])


def _matmul_z(h_ref, wr_ref, z_ref):
    z_ref[...] = jnp.dot(h_ref[...], wr_ref[...],
                         preferred_element_type=jnp.float32)


def _final_body(s0_ref, s1_ref, d0_ref, d1_ref, b_ref, z_ref, wl_ref,
                out_ref):
    mean_src = s0_ref[...] + s1_ref[...]
    dg = jnp.sum(d0_ref[...] + d1_ref[...], axis=1,
                 keepdims=True) * (1.0 / DWD)
    dg = jnp.maximum(dg, 1.0)
    mean = mean_src / dg
    pre = (jnp.dot(mean, wl_ref[...], preferred_element_type=jnp.float32)
           + b_ref[...] + z_ref[...])
    nrm = jnp.sqrt(jnp.sum(pre * pre, axis=1, keepdims=True))
    out_ref[...] = pre / jnp.maximum(nrm, 1e-12)


def _grid_call(body, in_specs, out_specs, out_shapes):
    return pl.pallas_call(
        body,
        grid=(NROW // BR,),
        in_specs=in_specs,
        out_specs=out_specs,
        out_shape=out_shapes,
    )


def kernel(features1, edge_index1,
           enc1_Wl, enc1_bl, enc1_Wr,
           enc2_Wl, enc2_bl, enc2_Wr,
           fc_W, fc_b,
           dec1_Wl, dec1_bl, dec1_Wr,
           dec2_Wl, dec2_bl, dec2_Wr):
    f32 = jnp.float32
    src = edge_index1[0].astype(jnp.int32)
    dst = edge_index1[1].astype(jnp.int32)
    pad = E_PAD - E
    pad_src = jnp.arange(pad, dtype=jnp.int32) % N
    pad_dst = N + (jnp.arange(pad, dtype=jnp.int32) % (NROW - N))
    src_p = jnp.concatenate([src, pad_src])
    dst_p = jnp.concatenate([dst, pad_dst])
    srcB = src_p.reshape(E_PAD // CH2, CH2)
    dstB = dst_p.reshape(E_PAD // CH2, CH2)
    x0 = jnp.concatenate([features1, jnp.zeros((NROW - N, D_IN), f32)])

    w1l = enc1_Wl.T.astype(f32)
    w1r = enc1_Wr.T.astype(f32)
    w2l = enc2_Wl.T.astype(f32)
    w2r = enc2_Wr.T.astype(f32)
    fcw = fc_W.T.astype(f32)
    w3l = dec1_Wl.T.astype(f32)
    w3r = dec1_Wr.T.astype(f32)
    w4l = dec2_Wl.T.astype(f32)
    w4r = dec2_Wr.T.astype(f32)
    b1 = enc1_bl.reshape(1, D_OUT)
    b2 = enc2_bl.reshape(1, D_OUT)
    fcb = fc_b.reshape(1, D_OUT)
    b3 = dec1_bl.reshape(1, D_OUT)
    b4 = dec2_bl.reshape(1, D_IN)

    degp = _degscatter(dstB)[0]
    d0, d1 = degp[:NROW], degp[NROW:]

    y1, z1 = _grid_call(
        _prep_body,
        [_row_spec(D_IN), _full_spec(D_IN, D_OUT), _full_spec(D_IN, D_OUT)],
        [_row_spec(D_OUT), _row_spec(D_OUT)],
        [jax.ShapeDtypeStruct((NROW, D_OUT), f32)] * 2,
    )(x0, w1l, w1r)

    sp = _segsum64(y1, srcB, dstB)[0]

    comb_in = [_row_spec(D_OUT), _row_spec(D_OUT), _row_spec(DWD),
               _row_spec(DWD), _full_spec(1, D_OUT), _row_spec(D_OUT)]
    y2 = _grid_call(
        _comb2_y,
        comb_in + [_full_spec(D_OUT, D_OUT)],
        _row_spec(D_OUT),
        jax.ShapeDtypeStruct((NROW, D_OUT), f32),
    )(sp[:NROW], sp[NROW:], d0, d1, b1, z1, w2l)
    z2 = _grid_call(
        _comb2_z,
        comb_in + [_full_spec(D_OUT, D_OUT)],
        _row_spec(D_OUT),
        jax.ShapeDtypeStruct((NROW, D_OUT), f32),
    )(sp[:NROW], sp[NROW:], d0, d1, b1, z1, w2r)

    sp2 = _segsum64(y2, srcB, dstB)[0]

    y3 = _grid_call(
        _comb_fc_y,
        comb_in + [_full_spec(D_OUT, D_OUT), _full_spec(1, D_OUT),
                   _full_spec(D_OUT, D_OUT)],
        _row_spec(D_OUT),
        jax.ShapeDtypeStruct((NROW, D_OUT), f32),
    )(sp2[:NROW], sp2[NROW:], d0, d1, b2, z2, fcw, fcb, w3l)
    x1f, z3 = _grid_call(
        _comb_fc_rest,
        comb_in + [_full_spec(D_OUT, D_OUT), _full_spec(1, D_OUT),
                   _full_spec(D_OUT, D_OUT)],
        [_row_spec(D_OUT), _row_spec(D_OUT)],
        [jax.ShapeDtypeStruct((NROW, D_OUT), f32)] * 2,
    )(sp2[:NROW], sp2[NROW:], d0, d1, b2, z2, fcw, fcb, w3r)

    sp3 = _segsum64(y3, srcB, dstB)[0]

    h2 = _grid_call(
        _comb_h2_h,
        comb_in,
        _row_spec(D_OUT),
        jax.ShapeDtypeStruct((NROW, D_OUT), f32),
    )(sp3[:NROW], sp3[NROW:], d0, d1, b3, z3)
    z4 = _grid_call(
        _matmul_z,
        [_row_spec(D_OUT), _full_spec(D_OUT, D_IN)],
        _row_spec(D_IN),
        jax.ShapeDtypeStruct((NROW, D_IN), f32),
    )(h2, w4r)

    sp4 = _segsum64(h2, srcB, dstB)[0]

    x1_rec = _grid_call(
        _final_body,
        [_row_spec(D_OUT), _row_spec(D_OUT), _row_spec(DWD),
         _row_spec(DWD), _full_spec(1, D_IN), _row_spec(D_IN),
         _full_spec(D_OUT, D_IN)],
        _row_spec(D_IN),
        jax.ShapeDtypeStruct((NROW, D_IN), f32),
    )(sp4[:NROW], sp4[NROW:], d0, d1, b4, z4, w4l)

    return (x1f[:N], x1_rec[:N])
